# Initial kernel scaffold; baseline (speedup 1.0000x reference)
import jax
import jax.numpy as jnp
from jax.experimental import pallas as pl


def kernel(x, local_edges_list, remote_edges_list, W1, b1, W2, b2):
    raise NotImplementedError("write your pallas kernel here")


# speedup vs baseline: 5.0826x; 5.0826x over previous
import functools

import jax
import jax.numpy as jnp
from jax import lax
from jax.experimental import pallas as pl
from jax.experimental.pallas import tpu as pltpu
from jax.experimental.pallas import tpu_sc as plsc

N = 10000
NPAD = 10240
DIN = 128
DHID = 256
NCLS = 40
EL = 288000
ER = 32000
NC = 2
NS = 16

CH = 48
SB = 24
ELP = NC * NS * 8 * SB * CH
ERP = NC * NS * 1 * SB * CH
LBH = ELP // (NS * SB * CH)
RBH = ERP // (NS * SB * CH)
PAD_DST = 10016


def _mesh():
    return plsc.VectorSubcoreMesh(core_axis_name="c", subcore_axis_name="s",
                                  num_cores=NC, num_subcores=NS)


def _newton_rsqrt(d):
    i = lax.bitcast_convert_type(d, jnp.int32)
    seed = jnp.full((16,), 0x5F3759DF, jnp.int32) - lax.shift_right_logical(i, 1)
    y = lax.bitcast_convert_type(seed, jnp.float32)
    for _ in range(3):
        y = y * (1.5 - 0.5 * d * y * y)
    return y


def _zero_fill(sbuf):
    def zf(r, _):
        for k in range(8):
            sbuf[r, pl.ds(16 * k, 16)] = jnp.zeros((16,), jnp.float32)
        return ()
    lax.fori_loop(0, 80, zf, ())


def _zero_acc(sid, sbuf, ag_sh):
    def zero(c, _):
        pltpu.sync_copy(sbuf, ag_sh.at[pl.ds(640 * sid + 80 * c, 80)])
        return ()
    lax.fori_loop(0, 8, zero, ())


def _edge_blocks(cid, sid, tab_h, ag_sh, ls_h, ld_h, rs_h, rd_h,
                 sidx, didx, idx1s, idx1d, rows):
    def blk(b, _):
        @pl.when(b < 8)
        def _():
            pltpu.sync_copy(ls_h.at[cid, sid, b], sidx)
            pltpu.sync_copy(ld_h.at[cid, sid, b], didx)

        @pl.when(b == 8)
        def _():
            pltpu.sync_copy(rs_h.at[cid, sid, 0], sidx)
            pltpu.sync_copy(rd_h.at[cid, sid, 0], didx)

        def body(i, _):
            for k in range(CH // 16):
                idx1s[pl.ds(16 * k, 16)] = sidx[i, pl.ds(16 * k, 16)]
                idx1d[pl.ds(16 * k, 16)] = didx[i, pl.ds(16 * k, 16)]
            pltpu.sync_copy(tab_h.at[idx1s], rows)
            pltpu.sync_copy(rows, ag_sh.at[idx1d], add=True)
            return ()
        lax.fori_loop(0, SB, body, ())
        return ()

    lax.fori_loop(0, 9, blk, ())


def _write_out(cid, sid, ag_sh, sbuf, out_h):
    lim = jnp.minimum(640 * (sid + 1), N) - 640 * sid
    nch = lim // 80

    def wout(c, _):
        r0 = 640 * sid + 80 * c
        pltpu.sync_copy(ag_sh.at[pl.ds(r0, 80)], sbuf)
        pltpu.sync_copy(sbuf, out_h.at[cid, pl.ds(r0, 80)])
        return ()
    lax.fori_loop(0, nch, wout, ())


@functools.lru_cache(maxsize=None)
def _build_sc_layer1():
  return pl.kernel(
    _sc_layer1_body,
    out_type=(
        jax.ShapeDtypeStruct((N, DIN), jnp.float32),
        jax.ShapeDtypeStruct((NC, N, DIN), jnp.float32),
        jax.ShapeDtypeStruct((NPAD,), jnp.float32),
    ),
    mesh=_mesh(),
    compiler_params=pltpu.CompilerParams(needs_layout_passes=False),
    scratch_types=[
        pltpu.VMEM_SHARED((NPAD, DIN), jnp.float32),
        pltpu.VMEM_SHARED((NPAD,), jnp.float32),
        pltpu.VMEM_SHARED((NPAD,), jnp.float32),
        pltpu.VMEM((SB, CH), jnp.int32),
        pltpu.VMEM((SB, CH), jnp.int32),
        pltpu.VMEM((CH,), jnp.int32),
        pltpu.VMEM((CH,), jnp.int32),
        pltpu.VMEM((CH, DIN), jnp.float32),
        pltpu.VMEM((80, DIN), jnp.float32),
        pltpu.VMEM((16, DIN), jnp.float32),
        pltpu.VMEM((16, DIN), jnp.float32),
        pltpu.VMEM((640,), jnp.float32),
        pltpu.VMEM((640,), jnp.float32),
        pltpu.VMEM((16,), jnp.float32),
        pltpu.VMEM((CH,), jnp.float32),
    ],
  )


def _sc_layer1_body(x_h, ldh_h, rdh_h, ls_h, ld_h, rs_h, rd_h,
                    xs_h, t1_h, dinv_h,
                    ag_sh, deg_sh, dinv_sh,
                    sidx, didx, idx1s, idx1d, rows, sbuf, xbuf, xsbuf,
                    degb, dinb, dvb, ones):
    cid = lax.axis_index("c")
    sid = lax.axis_index("s")

    _zero_fill(sbuf)
    _zero_acc(sid, sbuf, ag_sh)

    for k in range(CH // 16):
        ones[pl.ds(16 * k, 16)] = jnp.full((16,), 1.0, jnp.float32)
    for k in range(40):
        degb[pl.ds(16 * k, 16)] = jnp.full((16,), 1.0, jnp.float32)
    pltpu.sync_copy(degb, deg_sh.at[pl.ds(640 * sid, 640)])
    plsc.subcore_barrier()

    def hist(idx4_h, nblk):
        def blk(b, _):
            pltpu.sync_copy(idx4_h.at[sid, b], didx)

            def body(i, _):
                for k in range(CH // 16):
                    idx1d[pl.ds(16 * k, 16)] = didx[i, pl.ds(16 * k, 16)]
                pltpu.sync_copy(ones, deg_sh.at[idx1d], add=True)
                return ()
            lax.fori_loop(0, SB, body, ())
            return ()
        lax.fori_loop(0, nblk, blk, ())

    hist(ldh_h, LBH)
    hist(rdh_h, RBH)
    plsc.subcore_barrier()

    pltpu.sync_copy(deg_sh.at[pl.ds(640 * sid, 640)], degb)
    for k in range(40):
        dinb[pl.ds(16 * k, 16)] = _newton_rsqrt(degb[pl.ds(16 * k, 16)])
    pltpu.sync_copy(dinb, dinv_sh.at[pl.ds(640 * sid, 640)])

    @pl.when(cid == 0)
    def _():
        pltpu.sync_copy(dinb, dinv_h.at[pl.ds(640 * sid, 640)])

    plsc.subcore_barrier()

    def b_body(i, _):
        g = sid + NS * i

        @pl.when(g < 625)
        def _():
            r0 = g * 16
            pltpu.sync_copy(x_h.at[pl.ds(r0, 16)], xbuf)
            pltpu.sync_copy(dinv_sh.at[pl.ds(r0, 16)], dvb)
            for j in range(16):
                dv = plsc.load_gather(dvb, [jnp.full((16,), j, jnp.int32)])
                for k in range(DIN // 16):
                    xsbuf[j, pl.ds(16 * k, 16)] = (
                        xbuf[j, pl.ds(16 * k, 16)] * dv)
            pltpu.sync_copy(xsbuf, xs_h.at[pl.ds(r0, 16)])
        return ()

    lax.fori_loop(0, 40, b_body, ())
    plsc.subcore_barrier()

    _edge_blocks(cid, sid, xs_h, ag_sh, ls_h, ld_h, rs_h, rd_h,
                 sidx, didx, idx1s, idx1d, rows)
    plsc.subcore_barrier()

    _write_out(cid, sid, ag_sh, sbuf, t1_h)


@functools.lru_cache(maxsize=None)
def _build_sc_layer2():
  return pl.kernel(
    _sc_layer2_body,
    out_type=jax.ShapeDtypeStruct((NC, N, DIN), jnp.float32),
    mesh=_mesh(),
    compiler_params=pltpu.CompilerParams(needs_layout_passes=False),
    scratch_types=[
        pltpu.VMEM_SHARED((NPAD, DIN), jnp.float32),
        pltpu.VMEM((SB, CH), jnp.int32),
        pltpu.VMEM((SB, CH), jnp.int32),
        pltpu.VMEM((CH,), jnp.int32),
        pltpu.VMEM((CH,), jnp.int32),
        pltpu.VMEM((CH, DIN), jnp.float32),
        pltpu.VMEM((80, DIN), jnp.float32),
    ],
  )


def _sc_layer2_body(xs2_h, ls_h, ld_h, rs_h, rd_h, t2_h,
                    ag_sh, sidx, didx, idx1s, idx1d, rows, sbuf):
    cid = lax.axis_index("c")
    sid = lax.axis_index("s")

    _zero_fill(sbuf)
    _zero_acc(sid, sbuf, ag_sh)
    plsc.subcore_barrier()

    _edge_blocks(cid, sid, xs2_h, ag_sh, ls_h, ld_h, rs_h, rd_h,
                 sidx, didx, idx1s, idx1d, rows)
    plsc.subcore_barrier()

    _write_out(cid, sid, ag_sh, sbuf, t2_h)


def _tc_mlp_body(ta_ref, tb_ref, x_ref, dv_ref, w1_ref, b1_ref, w2_ref, o_ref):
    dv = dv_ref[...]
    a = (ta_ref[...] + tb_ref[...] + dv * x_ref[...]) * dv
    h = jnp.dot(a, w1_ref[...], preferred_element_type=jnp.float32)
    h = jnp.maximum(h + b1_ref[...], 0.0)
    g = jnp.dot(h, w2_ref[...], preferred_element_type=jnp.float32)
    o_ref[...] = g * dv


def _tc_out_body(ta_ref, tb_ref, xs2_ref, dv_ref, b2_ref, o_ref):
    s = (ta_ref[...] + tb_ref[...] + xs2_ref[...])[:, :NCLS]
    o = s * dv_ref[...] + b2_ref[...]
    m = jnp.max(o, axis=1, keepdims=True)
    e = o - m
    o_ref[...] = e - jnp.log(jnp.sum(jnp.exp(e), axis=1, keepdims=True))


_BM = 2000


def _pad_edges(row, fill, total):
    return jnp.concatenate(
        [row, jnp.full((total - row.shape[0],), fill, jnp.int32)])


def kernel(x, local_edges_list, remote_edges_list, W1, b1, W2, b2):
    lsp = _pad_edges(local_edges_list[0], 0, ELP)
    ldp = _pad_edges(local_edges_list[1], PAD_DST, ELP)
    rsp = _pad_edges(remote_edges_list[0], 0, ERP)
    rdp = _pad_edges(remote_edges_list[1], PAD_DST, ERP)

    ls2 = lsp.reshape(NC, NS, 8, SB, CH)
    ld2 = ldp.reshape(NC, NS, 8, SB, CH)
    rs2 = rsp.reshape(NC, NS, 1, SB, CH)
    rd2 = rdp.reshape(NC, NS, 1, SB, CH)
    ldh = ldp.reshape(NS, LBH, SB, CH)
    rdh = rdp.reshape(NS, RBH, SB, CH)

    xs1, t1, dinv_pad = _build_sc_layer1()(x, ldh, rdh, ls2, ld2, rs2, rd2)
    dinv = dinv_pad[:N].reshape(N, 1)

    xs2 = pl.pallas_call(
        _tc_mlp_body,
        grid=(N // _BM,),
        in_specs=[
            pl.BlockSpec((_BM, DIN), lambda i: (i, 0)),
            pl.BlockSpec((_BM, DIN), lambda i: (i, 0)),
            pl.BlockSpec((_BM, DIN), lambda i: (i, 0)),
            pl.BlockSpec((_BM, 1), lambda i: (i, 0)),
            pl.BlockSpec((DIN, DHID), lambda i: (0, 0)),
            pl.BlockSpec((1, DHID), lambda i: (0, 0)),
            pl.BlockSpec((DHID, DIN), lambda i: (0, 0)),
        ],
        out_specs=pl.BlockSpec((_BM, DIN), lambda i: (i, 0)),
        out_shape=jax.ShapeDtypeStruct((N, DIN), jnp.float32),
    )(t1[0], t1[1], x, dinv, W1, b1.reshape(1, DHID),
      jnp.pad(W2, ((0, 0), (0, DIN - NCLS))))

    t2 = _build_sc_layer2()(xs2, ls2, ld2, rs2, rd2)

    out = pl.pallas_call(
        _tc_out_body,
        grid=(N // _BM,),
        in_specs=[
            pl.BlockSpec((_BM, DIN), lambda i: (i, 0)),
            pl.BlockSpec((_BM, DIN), lambda i: (i, 0)),
            pl.BlockSpec((_BM, DIN), lambda i: (i, 0)),
            pl.BlockSpec((_BM, 1), lambda i: (i, 0)),
            pl.BlockSpec((1, NCLS), lambda i: (0, 0)),
        ],
        out_specs=pl.BlockSpec((_BM, NCLS), lambda i: (i, 0)),
        out_shape=jax.ShapeDtypeStruct((N, NCLS), jnp.float32),
    )(t2[0], t2[1], xs2, dinv, b2.reshape(1, NCLS))
    return out


# speedup vs baseline: 5.3493x; 1.0525x over previous
import functools

import jax
import jax.numpy as jnp
from jax import lax
from jax.experimental import pallas as pl
from jax.experimental.pallas import tpu as pltpu
from jax.experimental.pallas import tpu_sc as plsc

N = 10000
NPAD = 10240
DIN = 128
DHID = 256
NCLS = 40
EL = 288000
ER = 32000
NC = 2
NS = 16

CH = 48
SB = 24
ELP = NC * NS * 8 * SB * CH
ERP = NC * NS * 1 * SB * CH
LBH = ELP // (NS * SB * CH)
RBH = ERP // (NS * SB * CH)
PAD_DST = 10016


def _mesh():
    return plsc.VectorSubcoreMesh(core_axis_name="c", subcore_axis_name="s",
                                  num_cores=NC, num_subcores=NS)


def _newton_rsqrt(d):
    i = lax.bitcast_convert_type(d, jnp.int32)
    seed = jnp.full((16,), 0x5F3759DF, jnp.int32) - lax.shift_right_logical(i, 1)
    y = lax.bitcast_convert_type(seed, jnp.float32)
    for _ in range(3):
        y = y * (1.5 - 0.5 * d * y * y)
    return y


def _zero_fill(sbuf):
    def zf(r, _):
        for k in range(8):
            sbuf[r, pl.ds(16 * k, 16)] = jnp.zeros((16,), jnp.float32)
        return ()
    lax.fori_loop(0, 80, zf, ())


def _zero_acc(sid, sbuf, ag_sh):
    def zero(c, _):
        pltpu.sync_copy(sbuf, ag_sh.at[pl.ds(640 * sid + 80 * c, 80)])
        return ()
    lax.fori_loop(0, 8, zero, ())


def _edge_blocks(cid, sid, tab_h, ag_sh, ls_h, ld_h, rs_h, rd_h,
                 sidx, didx, idx1s, idx1d, rows, idx1s2, idx1d2, rows2,
                 sem, sem2):
    def blk(b, _):
        @pl.when(b < 8)
        def _():
            pltpu.sync_copy(ls_h.at[cid, sid, b], sidx)
            pltpu.sync_copy(ld_h.at[cid, sid, b], didx)

        @pl.when(b == 8)
        def _():
            pltpu.sync_copy(rs_h.at[cid, sid, 0], sidx)
            pltpu.sync_copy(rd_h.at[cid, sid, 0], didx)

        def body(j, _):
            i0 = 2 * j
            i1 = 2 * j + 1
            for k in range(CH // 16):
                idx1s[pl.ds(16 * k, 16)] = sidx[i0, pl.ds(16 * k, 16)]
                idx1d[pl.ds(16 * k, 16)] = didx[i0, pl.ds(16 * k, 16)]
                idx1s2[pl.ds(16 * k, 16)] = sidx[i1, pl.ds(16 * k, 16)]
                idx1d2[pl.ds(16 * k, 16)] = didx[i1, pl.ds(16 * k, 16)]
            d0 = pltpu.async_copy(tab_h.at[idx1s], rows, sem)
            d1 = pltpu.async_copy(tab_h.at[idx1s2], rows2, sem2)
            d0.wait()
            pltpu.sync_copy(rows, ag_sh.at[idx1d], add=True)
            d1.wait()
            pltpu.sync_copy(rows2, ag_sh.at[idx1d2], add=True)
            return ()
        lax.fori_loop(0, SB // 2, body, ())
        return ()

    lax.fori_loop(0, 9, blk, ())


def _write_out(cid, sid, ag_sh, sbuf, out_h):
    lim = jnp.minimum(640 * (sid + 1), N) - 640 * sid
    nch = lim // 80

    def wout(c, _):
        r0 = 640 * sid + 80 * c
        pltpu.sync_copy(ag_sh.at[pl.ds(r0, 80)], sbuf)
        pltpu.sync_copy(sbuf, out_h.at[cid, pl.ds(r0, 80)])
        return ()
    lax.fori_loop(0, nch, wout, ())


@functools.lru_cache(maxsize=None)
def _build_sc_layer1():
  return pl.kernel(
    _sc_layer1_body,
    out_type=(
        jax.ShapeDtypeStruct((N, DIN), jnp.float32),
        jax.ShapeDtypeStruct((NC, N, DIN), jnp.float32),
        jax.ShapeDtypeStruct((NPAD,), jnp.float32),
    ),
    mesh=_mesh(),
    compiler_params=pltpu.CompilerParams(needs_layout_passes=False),
    scratch_types=[
        pltpu.VMEM_SHARED((NPAD, DIN), jnp.float32),
        pltpu.VMEM_SHARED((NPAD,), jnp.float32),
        pltpu.VMEM_SHARED((NPAD,), jnp.float32),
        pltpu.VMEM((SB, CH), jnp.int32),
        pltpu.VMEM((SB, CH), jnp.int32),
        pltpu.VMEM((CH,), jnp.int32),
        pltpu.VMEM((CH,), jnp.int32),
        pltpu.VMEM((CH, DIN), jnp.float32),
        pltpu.VMEM((CH,), jnp.int32),
        pltpu.VMEM((CH,), jnp.int32),
        pltpu.VMEM((CH, DIN), jnp.float32),
        pltpu.SemaphoreType.DMA,
        pltpu.SemaphoreType.DMA,
        pltpu.VMEM((80, DIN), jnp.float32),
        pltpu.VMEM((16, DIN), jnp.float32),
        pltpu.VMEM((16, DIN), jnp.float32),
        pltpu.VMEM((640,), jnp.float32),
        pltpu.VMEM((640,), jnp.float32),
        pltpu.VMEM((16,), jnp.float32),
        pltpu.VMEM((CH,), jnp.float32),
    ],
  )


def _sc_layer1_body(x_h, ldh_h, rdh_h, ls_h, ld_h, rs_h, rd_h,
                    xs_h, t1_h, dinv_h,
                    ag_sh, deg_sh, dinv_sh,
                    sidx, didx, idx1s, idx1d, rows, idx1s2, idx1d2, rows2,
                    sem, sem2, sbuf, xbuf, xsbuf,
                    degb, dinb, dvb, ones):
    cid = lax.axis_index("c")
    sid = lax.axis_index("s")

    _zero_fill(sbuf)
    _zero_acc(sid, sbuf, ag_sh)

    for k in range(CH // 16):
        ones[pl.ds(16 * k, 16)] = jnp.full((16,), 1.0, jnp.float32)
    for k in range(40):
        degb[pl.ds(16 * k, 16)] = jnp.full((16,), 1.0, jnp.float32)
    pltpu.sync_copy(degb, deg_sh.at[pl.ds(640 * sid, 640)])
    plsc.subcore_barrier()

    def hist(idx4_h, nblk):
        def blk(b, _):
            pltpu.sync_copy(idx4_h.at[sid, b], didx)

            def body(i, _):
                for k in range(CH // 16):
                    idx1d[pl.ds(16 * k, 16)] = didx[i, pl.ds(16 * k, 16)]
                pltpu.sync_copy(ones, deg_sh.at[idx1d], add=True)
                return ()
            lax.fori_loop(0, SB, body, ())
            return ()
        lax.fori_loop(0, nblk, blk, ())

    hist(ldh_h, LBH)
    hist(rdh_h, RBH)
    plsc.subcore_barrier()

    pltpu.sync_copy(deg_sh.at[pl.ds(640 * sid, 640)], degb)
    for k in range(40):
        dinb[pl.ds(16 * k, 16)] = _newton_rsqrt(degb[pl.ds(16 * k, 16)])
    pltpu.sync_copy(dinb, dinv_sh.at[pl.ds(640 * sid, 640)])

    @pl.when(cid == 0)
    def _():
        pltpu.sync_copy(dinb, dinv_h.at[pl.ds(640 * sid, 640)])

    plsc.subcore_barrier()

    def b_body(i, _):
        g = sid + NS * i

        @pl.when(g < 625)
        def _():
            r0 = g * 16
            pltpu.sync_copy(x_h.at[pl.ds(r0, 16)], xbuf)
            pltpu.sync_copy(dinv_sh.at[pl.ds(r0, 16)], dvb)
            for j in range(16):
                dv = plsc.load_gather(dvb, [jnp.full((16,), j, jnp.int32)])
                for k in range(DIN // 16):
                    xsbuf[j, pl.ds(16 * k, 16)] = (
                        xbuf[j, pl.ds(16 * k, 16)] * dv)
            pltpu.sync_copy(xsbuf, xs_h.at[pl.ds(r0, 16)])
        return ()

    lax.fori_loop(0, 40, b_body, ())
    plsc.subcore_barrier()

    _edge_blocks(cid, sid, xs_h, ag_sh, ls_h, ld_h, rs_h, rd_h,
                 sidx, didx, idx1s, idx1d, rows, idx1s2, idx1d2, rows2,
                 sem, sem2)
    plsc.subcore_barrier()

    _write_out(cid, sid, ag_sh, sbuf, t1_h)


@functools.lru_cache(maxsize=None)
def _build_sc_layer2():
  return pl.kernel(
    _sc_layer2_body,
    out_type=jax.ShapeDtypeStruct((NC, N, DIN), jnp.float32),
    mesh=_mesh(),
    compiler_params=pltpu.CompilerParams(needs_layout_passes=False),
    scratch_types=[
        pltpu.VMEM_SHARED((NPAD, DIN), jnp.float32),
        pltpu.VMEM((SB, CH), jnp.int32),
        pltpu.VMEM((SB, CH), jnp.int32),
        pltpu.VMEM((CH,), jnp.int32),
        pltpu.VMEM((CH,), jnp.int32),
        pltpu.VMEM((CH, DIN), jnp.float32),
        pltpu.VMEM((CH,), jnp.int32),
        pltpu.VMEM((CH,), jnp.int32),
        pltpu.VMEM((CH, DIN), jnp.float32),
        pltpu.SemaphoreType.DMA,
        pltpu.SemaphoreType.DMA,
        pltpu.VMEM((80, DIN), jnp.float32),
    ],
  )


def _sc_layer2_body(xs2_h, ls_h, ld_h, rs_h, rd_h, t2_h,
                    ag_sh, sidx, didx, idx1s, idx1d, rows,
                    idx1s2, idx1d2, rows2, sem, sem2, sbuf):
    cid = lax.axis_index("c")
    sid = lax.axis_index("s")

    _zero_fill(sbuf)
    _zero_acc(sid, sbuf, ag_sh)
    plsc.subcore_barrier()

    _edge_blocks(cid, sid, xs2_h, ag_sh, ls_h, ld_h, rs_h, rd_h,
                 sidx, didx, idx1s, idx1d, rows, idx1s2, idx1d2, rows2,
                 sem, sem2)
    plsc.subcore_barrier()

    _write_out(cid, sid, ag_sh, sbuf, t2_h)


def _tc_mlp_body(ta_ref, tb_ref, x_ref, dv_ref, w1_ref, b1_ref, w2_ref, o_ref):
    dv = dv_ref[...]
    a = (ta_ref[...] + tb_ref[...] + dv * x_ref[...]) * dv
    h = jnp.dot(a, w1_ref[...], preferred_element_type=jnp.float32)
    h = jnp.maximum(h + b1_ref[...], 0.0)
    g = jnp.dot(h, w2_ref[...], preferred_element_type=jnp.float32)
    o_ref[...] = g * dv


def _tc_out_body(ta_ref, tb_ref, xs2_ref, dv_ref, b2_ref, o_ref):
    s = (ta_ref[...] + tb_ref[...] + xs2_ref[...])[:, :NCLS]
    o = s * dv_ref[...] + b2_ref[...]
    m = jnp.max(o, axis=1, keepdims=True)
    e = o - m
    o_ref[...] = e - jnp.log(jnp.sum(jnp.exp(e), axis=1, keepdims=True))


_BM = 2000


def _pad_edges(row, fill, total):
    return jnp.concatenate(
        [row, jnp.full((total - row.shape[0],), fill, jnp.int32)])


def kernel(x, local_edges_list, remote_edges_list, W1, b1, W2, b2):
    lsp = _pad_edges(local_edges_list[0], 0, ELP)
    ldp = _pad_edges(local_edges_list[1], PAD_DST, ELP)
    rsp = _pad_edges(remote_edges_list[0], 0, ERP)
    rdp = _pad_edges(remote_edges_list[1], PAD_DST, ERP)

    ls2 = lsp.reshape(NC, NS, 8, SB, CH)
    ld2 = ldp.reshape(NC, NS, 8, SB, CH)
    rs2 = rsp.reshape(NC, NS, 1, SB, CH)
    rd2 = rdp.reshape(NC, NS, 1, SB, CH)
    ldh = ldp.reshape(NS, LBH, SB, CH)
    rdh = rdp.reshape(NS, RBH, SB, CH)

    xs1, t1, dinv_pad = _build_sc_layer1()(x, ldh, rdh, ls2, ld2, rs2, rd2)
    dinv = dinv_pad[:N].reshape(N, 1)

    xs2 = pl.pallas_call(
        _tc_mlp_body,
        grid=(N // _BM,),
        in_specs=[
            pl.BlockSpec((_BM, DIN), lambda i: (i, 0)),
            pl.BlockSpec((_BM, DIN), lambda i: (i, 0)),
            pl.BlockSpec((_BM, DIN), lambda i: (i, 0)),
            pl.BlockSpec((_BM, 1), lambda i: (i, 0)),
            pl.BlockSpec((DIN, DHID), lambda i: (0, 0)),
            pl.BlockSpec((1, DHID), lambda i: (0, 0)),
            pl.BlockSpec((DHID, DIN), lambda i: (0, 0)),
        ],
        out_specs=pl.BlockSpec((_BM, DIN), lambda i: (i, 0)),
        out_shape=jax.ShapeDtypeStruct((N, DIN), jnp.float32),
    )(t1[0], t1[1], x, dinv, W1, b1.reshape(1, DHID),
      jnp.pad(W2, ((0, 0), (0, DIN - NCLS))))

    t2 = _build_sc_layer2()(xs2, ls2, ld2, rs2, rd2)

    out = pl.pallas_call(
        _tc_out_body,
        grid=(N // _BM,),
        in_specs=[
            pl.BlockSpec((_BM, DIN), lambda i: (i, 0)),
            pl.BlockSpec((_BM, DIN), lambda i: (i, 0)),
            pl.BlockSpec((_BM, DIN), lambda i: (i, 0)),
            pl.BlockSpec((_BM, 1), lambda i: (i, 0)),
            pl.BlockSpec((1, NCLS), lambda i: (0, 0)),
        ],
        out_specs=pl.BlockSpec((_BM, NCLS), lambda i: (i, 0)),
        out_shape=jax.ShapeDtypeStruct((N, NCLS), jnp.float32),
    )(t2[0], t2[1], xs2, dinv, b2.reshape(1, NCLS))
    return out
